# Initial kernel scaffold; baseline (speedup 1.0000x reference)
import jax
import jax.numpy as jnp
from jax.experimental import pallas as pl


def kernel(input_ids, scores, allowed_ids, next_for_ids):
    raise NotImplementedError("write your pallas kernel here")


# speedup vs baseline: 3.0624x; 3.0624x over previous
import functools

import jax
import jax.numpy as jnp
from jax import lax
from jax.experimental import pallas as pl
from jax.experimental.pallas import tpu as pltpu
from jax.experimental.pallas import tpu_sc as plsc

EDGE_BOOST_VAL = 3.0
L = 16


def _make_sc_kernel(B, V, K, NF):
    NC, NS = 2, 16
    NW = NC * NS
    ROWS = B // NW
    KC = K // 128
    mesh = plsc.VectorSubcoreMesh(core_axis_name="c", subcore_axis_name="s")

    @functools.partial(
        pl.kernel,
        mesh=mesh,
        compiler_params=pltpu.CompilerParams(needs_layout_passes=False),
        out_type=jax.ShapeDtypeStruct((B, V), jnp.float32),
        scratch_types=[
            pltpu.VMEM((V,), jnp.float32),
            pltpu.VMEM((KC, 128), jnp.int32),
            pltpu.VMEM((KC, 128), jnp.int32),
            pltpu.VMEM((KC, 128), jnp.float32),
            pltpu.VMEM((NF,), jnp.int32),
            pltpu.VMEM((NF,), jnp.float32),
            pltpu.SemaphoreType.DMA,
        ],
    )
    def k(scores_hbm, allowed_hbm, nf_hbm, out_hbm,
          row_v, ids_v, fidx_v, vals_v, nf_v, bval_v, sem):
        wid = lax.axis_index("s") * NC + lax.axis_index("c")

        pltpu.sync_copy(nf_hbm, nf_v)
        iota = lax.iota(jnp.int32, L)
        for j in range(NF // L):
            idx = nf_v[pl.ds(j * L, L)]
            tag = (iota + (j * L + 1)).astype(jnp.float32)
            plsc.store_scatter(row_v, [idx], tag)
        for j in range(NF // L):
            idx = nf_v[pl.ds(j * L, L)]
            tag = (iota + (j * L + 1)).astype(jnp.float32)
            back = plsc.load_gather(row_v, [idx])
            bval = jnp.where(back == tag, jnp.float32(EDGE_BOOST_VAL),
                             jnp.float32(0.0))
            bval_v[pl.ds(j * L, L)] = bval

        neg_inf = jnp.full((L,), float("-inf"), jnp.float32)

        for r in range(ROWS):
            b = wid * ROWS + r
            pltpu.sync_copy(allowed_hbm.at[b], ids_v)
            base = b * V
            for j in range(KC):
                for i in range(128 // L):
                    ids = ids_v[j, pl.ds(i * L, L)]
                    fidx_v[j, pl.ds(i * L, L)] = ids + base
            copies = [
                pltpu.async_copy(scores_hbm.at[fidx_v.at[j]], vals_v.at[j], sem)
                for j in range(KC)
            ]

            def fill_body(i, _):
                for u in range(4):
                    row_v[pl.ds(i * (4 * L) + u * L, L)] = neg_inf
                return 0

            lax.fori_loop(0, V // (4 * L), fill_body, 0, unroll=2)

            for c in copies:
                c.wait()

            for j in range(KC):
                for i in range(128 // L):
                    idx = ids_v[j, pl.ds(i * L, L)]
                    val = vals_v[j, pl.ds(i * L, L)]
                    plsc.store_scatter(row_v, [idx], val)
            for j in range(NF // L):
                idx = nf_v[pl.ds(j * L, L)]
                bv = bval_v[pl.ds(j * L, L)]
                plsc.addupdate_scatter(row_v, [idx], bv)

            pltpu.sync_copy(row_v, out_hbm.at[b])

    return k


def kernel(input_ids, scores, allowed_ids, next_for_ids):
    del input_ids
    B, V = scores.shape
    K = allowed_ids.shape[1]
    NF = next_for_ids.shape[0]
    k = _make_sc_kernel(B, V, K, NF)
    return k(scores.reshape(-1),
             allowed_ids.reshape(B, K // 128, 128).astype(jnp.int32),
             next_for_ids.astype(jnp.int32))


# speedup vs baseline: 4.6897x; 1.5314x over previous
import functools

import jax
import jax.numpy as jnp
from jax import lax
from jax.experimental import pallas as pl
from jax.experimental.pallas import tpu as pltpu
from jax.experimental.pallas import tpu_sc as plsc

EDGE_BOOST_VAL = 3.0
L = 16


def _make_sc_kernel(B, V, K, NF):
    NC, NS = 2, 16
    NW = NC * NS
    ROWS = B // NW
    KC = K // 128
    C = 33280
    OFFS = (0, C, 2 * C)
    SIZES = (C, C, V - 2 * C)
    mesh = plsc.VectorSubcoreMesh(core_axis_name="c", subcore_axis_name="s")

    @functools.partial(
        pl.kernel,
        mesh=mesh,
        compiler_params=pltpu.CompilerParams(needs_layout_passes=False),
        out_type=jax.ShapeDtypeStruct((B, V), jnp.float32),
        scratch_types=[
            pltpu.VMEM((SIZES[0],), jnp.float32),
            pltpu.VMEM((SIZES[1],), jnp.float32),
            pltpu.VMEM((SIZES[2],), jnp.float32),
            pltpu.VMEM((ROWS, KC, 128), jnp.int32),
            pltpu.VMEM((KC, 128), jnp.float32),
            pltpu.VMEM((NF,), jnp.int32),
            pltpu.VMEM((NF,), jnp.float32),
            pltpu.SemaphoreType.DMA,
            pltpu.SemaphoreType.DMA,
            pltpu.SemaphoreType.DMA,
            pltpu.SemaphoreType.DMA,
            pltpu.SemaphoreType.DMA,
            pltpu.SemaphoreType.DMA,
        ],
    )
    def k(scores_hbm, allowed_hbm, nf_hbm, out_hbm,
          buf0, buf1, buf2, ids_v, vals_v, nf_v, bval_v,
          isem0, isem1, isem2, osem0, osem1, osem2):
        wid = lax.axis_index("s") * NC + lax.axis_index("c")
        bufs = [buf0, buf1, buf2]
        isems = [isem0, isem1, isem2]
        osems = [osem0, osem1, osem2]

        pltpu.sync_copy(nf_hbm, nf_v)
        iota = lax.iota(jnp.int32, L)
        for j in range(NF // L):
            idx = nf_v[pl.ds(j * L, L)]
            tag = (iota + (j * L + 1)).astype(jnp.float32)
            for h in range(3):
                m = (idx >= OFFS[h]) & (idx < OFFS[h] + SIZES[h])
                plsc.store_scatter(bufs[h], [idx - OFFS[h]], tag, mask=m)
        for j in range(NF // L):
            idx = nf_v[pl.ds(j * L, L)]
            tag = (iota + (j * L + 1)).astype(jnp.float32)
            back = jnp.zeros((L,), jnp.float32)
            for h in range(3):
                m = (idx >= OFFS[h]) & (idx < OFFS[h] + SIZES[h])
                g = plsc.load_gather(bufs[h],
                                     [jnp.where(m, idx - OFFS[h], 0)])
                back = jnp.where(m, g, back)
            bval_v[pl.ds(j * L, L)] = jnp.where(
                back == tag, jnp.float32(EDGE_BOOST_VAL), jnp.float32(0.0))

        for r in range(ROWS):
            b = wid * ROWS + r
            pltpu.sync_copy(allowed_hbm.at[b], ids_v.at[r])

        neg_inf = jnp.full((L,), float("-inf"), jnp.float32)
        chunks = [(r, h) for r in range(ROWS) for h in range(3)]
        in_copies = [None, None, None]
        out_copies = [None, None, None]

        def fire_in(c):
            r, h = chunks[c]
            b = wid * ROWS + r
            in_copies[h] = pltpu.async_copy(
                scores_hbm.at[b].at[pl.ds(OFFS[h], SIZES[h])],
                bufs[h], isems[h])

        fire_in(0)
        for c, (r, h) in enumerate(chunks):
            b = wid * ROWS + r
            lo, hsz = OFFS[h], SIZES[h]
            buf = bufs[h]

            if c + 1 < len(chunks):
                h2 = chunks[c + 1][1]
                if out_copies[h2] is not None:
                    out_copies[h2].wait()
                fire_in(c + 1)

            in_copies[h].wait()

            for j in range(KC):
                for i in range(128 // L):
                    ids = ids_v[r, j, pl.ds(i * L, L)]
                    m = (ids >= lo) & (ids < lo + hsz)
                    gidx = jnp.where(m, ids - lo, 0)
                    vals_v[j, pl.ds(i * L, L)] = plsc.load_gather(buf, [gidx])

            def fill_body(i, _):
                buf[pl.ds(i * L, L)] = neg_inf
                return 0
            lax.fori_loop(0, hsz // L, fill_body, 0, unroll=8)

            for j in range(KC):
                for i in range(128 // L):
                    ids = ids_v[r, j, pl.ds(i * L, L)]
                    val = vals_v[j, pl.ds(i * L, L)]
                    m = (ids >= lo) & (ids < lo + hsz)
                    plsc.store_scatter(buf, [ids - lo], val, mask=m)
            for j in range(NF // L):
                idx = nf_v[pl.ds(j * L, L)]
                bv = bval_v[pl.ds(j * L, L)]
                m = (idx >= lo) & (idx < lo + hsz)
                plsc.addupdate_scatter(buf, [idx - lo], bv, mask=m)

            out_copies[h] = pltpu.async_copy(
                buf, out_hbm.at[b].at[pl.ds(lo, hsz)], osems[h])

        for h in range(3):
            if out_copies[h] is not None:
                out_copies[h].wait()

    return k


def kernel(input_ids, scores, allowed_ids, next_for_ids):
    del input_ids
    B, V = scores.shape
    K = allowed_ids.shape[1]
    NF = next_for_ids.shape[0]
    k = _make_sc_kernel(B, V, K, NF)
    return k(scores,
             allowed_ids.reshape(B, K // 128, 128),
             next_for_ids)


# speedup vs baseline: 5.0101x; 1.0683x over previous
import functools

import jax
import jax.numpy as jnp
from jax import lax
from jax.experimental import pallas as pl
from jax.experimental.pallas import tpu as pltpu
from jax.experimental.pallas import tpu_sc as plsc

EDGE_BOOST_VAL = 3.0
L = 16


def _make_sc_kernel(B, V, K, NF):
    NC, NS = 2, 16
    NW = NC * NS
    ROWS = B // NW
    C = 33280
    OFFS = (0, C, 2 * C)
    SIZES = (C, C, V - 2 * C)
    mesh = plsc.VectorSubcoreMesh(core_axis_name="c", subcore_axis_name="s")

    @functools.partial(
        pl.kernel,
        mesh=mesh,
        compiler_params=pltpu.CompilerParams(needs_layout_passes=False),
        out_type=jax.ShapeDtypeStruct((B, V), jnp.float32),
        scratch_types=[
            pltpu.VMEM((SIZES[0],), jnp.float32),
            pltpu.VMEM((SIZES[1],), jnp.float32),
            pltpu.VMEM((SIZES[2],), jnp.float32),
            pltpu.VMEM((ROWS, K), jnp.int32),
            pltpu.VMEM((K,), jnp.float32),
            pltpu.VMEM((NF,), jnp.int32),
            pltpu.VMEM((NF,), jnp.float32),
            pltpu.SemaphoreType.DMA,
            pltpu.SemaphoreType.DMA,
            pltpu.SemaphoreType.DMA,
            pltpu.SemaphoreType.DMA,
            pltpu.SemaphoreType.DMA,
            pltpu.SemaphoreType.DMA,
        ],
    )
    def k(scores_hbm, allowed_hbm, nf_hbm, out_hbm,
          buf0, buf1, buf2, ids_v, vals_v, nf_v, bval_v,
          isem0, isem1, isem2, osem0, osem1, osem2):
        wid = lax.axis_index("s") * NC + lax.axis_index("c")
        bufs = [buf0, buf1, buf2]
        isems = [isem0, isem1, isem2]
        osems = [osem0, osem1, osem2]

        pltpu.sync_copy(nf_hbm, nf_v)
        iota = lax.iota(jnp.int32, L)
        for j in range(NF // L):
            idx = nf_v[pl.ds(j * L, L)]
            tag = (iota + (j * L + 1)).astype(jnp.float32)
            for h in range(3):
                m = (idx >= OFFS[h]) & (idx < OFFS[h] + SIZES[h])
                plsc.store_scatter(bufs[h], [idx - OFFS[h]], tag, mask=m)
        for j in range(NF // L):
            idx = nf_v[pl.ds(j * L, L)]
            tag = (iota + (j * L + 1)).astype(jnp.float32)
            back = jnp.zeros((L,), jnp.float32)
            for h in range(3):
                m = (idx >= OFFS[h]) & (idx < OFFS[h] + SIZES[h])
                g = plsc.load_gather(bufs[h],
                                     [jnp.where(m, idx - OFFS[h], 0)])
                back = jnp.where(m, g, back)
            bval_v[pl.ds(j * L, L)] = jnp.where(
                back == tag, jnp.float32(EDGE_BOOST_VAL), jnp.float32(0.0))

        for r in range(ROWS):
            b = wid * ROWS + r
            pltpu.sync_copy(allowed_hbm.at[b], ids_v.at[r])

        neg_inf = jnp.full((L,), float("-inf"), jnp.float32)
        chunks = [(r, h) for r in range(ROWS) for h in range(3)]
        in_copies = [None, None, None]
        out_copies = [None, None, None]

        def fire_in(c):
            r, h = chunks[c]
            b = wid * ROWS + r
            in_copies[h] = pltpu.async_copy(
                scores_hbm.at[b].at[pl.ds(OFFS[h], SIZES[h])],
                bufs[h], isems[h])

        fire_in(0)
        for c, (r, h) in enumerate(chunks):
            b = wid * ROWS + r
            lo, hsz = OFFS[h], SIZES[h]
            buf = bufs[h]

            if c + 1 < len(chunks):
                h2 = chunks[c + 1][1]
                if out_copies[h2] is not None:
                    out_copies[h2].wait()
                fire_in(c + 1)

            in_copies[h].wait()

            def gather_body(j, _):
                ids = ids_v[r, pl.ds(j * L, L)]
                m = (ids >= lo) & (ids < lo + hsz)
                gidx = jnp.where(m, ids - lo, 0)
                vals_v[pl.ds(j * L, L)] = plsc.load_gather(buf, [gidx])
                return 0
            lax.fori_loop(0, K // L, gather_body, 0)

            def fill_body(i, _):
                buf[pl.ds(i * L, L)] = neg_inf
                return 0
            lax.fori_loop(0, hsz // L, fill_body, 0, unroll=8)

            def scatter_body(j, _):
                ids = ids_v[r, pl.ds(j * L, L)]
                val = vals_v[pl.ds(j * L, L)]
                m = (ids >= lo) & (ids < lo + hsz)
                plsc.store_scatter(buf, [ids - lo], val, mask=m)
                return 0
            lax.fori_loop(0, K // L, scatter_body, 0)

            def boost_body(j, _):
                idx = nf_v[pl.ds(j * L, L)]
                bv = bval_v[pl.ds(j * L, L)]
                m = (idx >= lo) & (idx < lo + hsz)
                plsc.addupdate_scatter(buf, [idx - lo], bv, mask=m)
                return 0
            lax.fori_loop(0, NF // L, boost_body, 0)

            out_copies[h] = pltpu.async_copy(
                buf, out_hbm.at[b].at[pl.ds(lo, hsz)], osems[h])

        for h in range(3):
            if out_copies[h] is not None:
                out_copies[h].wait()

    return k


def kernel(input_ids, scores, allowed_ids, next_for_ids):
    del input_ids
    B, V = scores.shape
    K = allowed_ids.shape[1]
    NF = next_for_ids.shape[0]
    k = _make_sc_kernel(B, V, K, NF)
    return k(scores, allowed_ids, next_for_ids)


# speedup vs baseline: 5.0844x; 1.0148x over previous
import functools

import jax
import jax.numpy as jnp
from jax import lax
from jax.experimental import pallas as pl
from jax.experimental.pallas import tpu as pltpu
from jax.experimental.pallas import tpu_sc as plsc

EDGE_BOOST_VAL = 3.0
L = 16


def _make_sc_kernel(B, V, K, NF):
    NC, NS = 2, 16
    NW = NC * NS
    ROWS = B // NW
    C = 33280
    OFFS = (0, C, 2 * C)
    SIZES = (C, C, V - 2 * C)
    HORDER = (0, 1, 2)
    mesh = plsc.VectorSubcoreMesh(core_axis_name="c", subcore_axis_name="s")

    @functools.partial(
        pl.kernel,
        mesh=mesh,
        compiler_params=pltpu.CompilerParams(needs_layout_passes=False),
        out_type=jax.ShapeDtypeStruct((B, V), jnp.float32),
        scratch_types=[
            pltpu.VMEM((SIZES[0],), jnp.float32),
            pltpu.VMEM((SIZES[1],), jnp.float32),
            pltpu.VMEM((SIZES[2],), jnp.float32),
            pltpu.VMEM((ROWS, K), jnp.int32),
            pltpu.VMEM((K,), jnp.float32),
            pltpu.VMEM((NF,), jnp.int32),
            pltpu.VMEM((NF,), jnp.float32),
            pltpu.SemaphoreType.DMA,
            pltpu.SemaphoreType.DMA,
            pltpu.SemaphoreType.DMA,
            pltpu.SemaphoreType.DMA,
            pltpu.SemaphoreType.DMA,
            pltpu.SemaphoreType.DMA,
            pltpu.SemaphoreType.DMA,
        ],
    )
    def k(scores_hbm, allowed_hbm, nf_hbm, out_hbm,
          buf0, buf1, buf2, ids_v, vals_v, nf_v, bval_v,
          idsem, isem0, isem1, isem2, osem0, osem1, osem2):
        wid = lax.axis_index("s") * NC + lax.axis_index("c")
        bufs = [buf0, buf1, buf2]
        isems = [isem0, isem1, isem2]
        osems = [osem0, osem1, osem2]

        chunks = [(r, h) for r in range(ROWS) for h in HORDER]
        in_copies = [None, None, None]
        out_copies = [None, None, None]

        def fire_in(c):
            r, h = chunks[c]
            b = wid * ROWS + r
            in_copies[h] = pltpu.async_copy(
                scores_hbm.at[b].at[pl.ds(OFFS[h], SIZES[h])],
                bufs[h], isems[h])

        fire_in(0)
        fire_in(1)
        id_copies = [
            pltpu.async_copy(allowed_hbm.at[wid * ROWS + r], ids_v.at[r], idsem)
            for r in range(ROWS)
        ]

        pltpu.sync_copy(nf_hbm, nf_v)
        iota = lax.iota(jnp.int32, L)
        for h in range(3):
            for j in range(NF // L):
                idx = nf_v[pl.ds(j * L, L)]
                tag = (iota + (j * L + 1)).astype(jnp.float32)
                m = (idx >= OFFS[h]) & (idx < OFFS[h] + SIZES[h])
                plsc.store_scatter(buf2, [jnp.where(m, idx - OFFS[h], 0)],
                                   tag, mask=m)
            for j in range(NF // L):
                idx = nf_v[pl.ds(j * L, L)]
                tag = (iota + (j * L + 1)).astype(jnp.float32)
                m = (idx >= OFFS[h]) & (idx < OFFS[h] + SIZES[h])
                back = plsc.load_gather(buf2, [jnp.where(m, idx - OFFS[h], 0)])
                bv = jnp.where(m & (back == tag),
                               jnp.float32(EDGE_BOOST_VAL), jnp.float32(0.0))
                if h == 0:
                    bval_v[pl.ds(j * L, L)] = bv
                else:
                    bval_v[pl.ds(j * L, L)] = bval_v[pl.ds(j * L, L)] + bv

        fire_in(2)
        for cp in id_copies:
            cp.wait()

        neg_inf = jnp.full((L,), float("-inf"), jnp.float32)
        for c, (r, h) in enumerate(chunks):
            b = wid * ROWS + r
            lo, hsz = OFFS[h], SIZES[h]
            buf = bufs[h]

            cn = c + 2
            if 3 <= cn < len(chunks):
                hn = chunks[cn][1]
                if out_copies[hn] is not None:
                    out_copies[hn].wait()
                fire_in(cn)

            in_copies[h].wait()

            def gather_body(j, _):
                ids = ids_v[r, pl.ds(j * L, L)]
                m = (ids >= lo) & (ids < lo + hsz)
                gidx = jnp.where(m, ids - lo, 0)
                vals_v[pl.ds(j * L, L)] = plsc.load_gather(buf, [gidx])
                return 0
            lax.fori_loop(0, K // L, gather_body, 0)

            def fill_body(i, _):
                buf[pl.ds(i * L, L)] = neg_inf
                return 0
            lax.fori_loop(0, hsz // L, fill_body, 0, unroll=8)

            def scatter_body(j, _):
                ids = ids_v[r, pl.ds(j * L, L)]
                val = vals_v[pl.ds(j * L, L)]
                m = (ids >= lo) & (ids < lo + hsz)
                plsc.store_scatter(buf, [ids - lo], val, mask=m)
                return 0
            lax.fori_loop(0, K // L, scatter_body, 0)

            def boost_body(j, _):
                idx = nf_v[pl.ds(j * L, L)]
                bv = bval_v[pl.ds(j * L, L)]
                m = (idx >= lo) & (idx < lo + hsz)
                plsc.addupdate_scatter(buf, [idx - lo], bv, mask=m)
                return 0
            lax.fori_loop(0, NF // L, boost_body, 0)

            out_copies[h] = pltpu.async_copy(
                buf, out_hbm.at[b].at[pl.ds(lo, hsz)], osems[h])

        for h in range(3):
            if out_copies[h] is not None:
                out_copies[h].wait()

    return k


def kernel(input_ids, scores, allowed_ids, next_for_ids):
    del input_ids
    B, V = scores.shape
    K = allowed_ids.shape[1]
    NF = next_for_ids.shape[0]
    k = _make_sc_kernel(B, V, K, NF)
    return k(scores, allowed_ids, next_for_ids)
